# baseline (speedup 1.0000x reference)
import functools

import jax
import jax.numpy as jnp
from jax import lax
from jax.experimental import pallas as pl
from jax.experimental.pallas import tpu as pltpu
from jax.experimental.pallas import tpu_sc as plsc

_CHUNK = 128


def _sc_gather(didx, ridx, dtab16, rtab16):
    B = didx.shape[0]
    EMB = dtab16.shape[1]
    info = plsc.get_sparse_core_info()
    NC, NS = info.num_cores, info.num_subcores
    NW = NC * NS
    b_per_w = B // NW
    n_chunks = b_per_w // _CHUNK

    didx2 = didx.reshape(B // _CHUNK, _CHUNK)
    ridx2 = ridx.reshape(B // _CHUNK, _CHUNK)

    mesh = plsc.VectorSubcoreMesh(core_axis_name="c", subcore_axis_name="s")

    @functools.partial(
        pl.kernel,
        mesh=mesh,
        compiler_params=pltpu.CompilerParams(use_tc_tiling_on_sc=False),
        out_type=[
            jax.ShapeDtypeStruct((B, EMB), jnp.bfloat16),
            jax.ShapeDtypeStruct((B, EMB), jnp.bfloat16),
        ],
        scratch_types=[
            pltpu.VMEM((n_chunks, _CHUNK), jnp.int32),
            pltpu.VMEM((n_chunks, _CHUNK), jnp.int32),
            pltpu.VMEM((b_per_w, EMB), jnp.bfloat16),
            pltpu.VMEM((b_per_w, EMB), jnp.bfloat16),
            pltpu.SemaphoreType.DMA,
        ],
    )
    def gather_kernel(didx_hbm, ridx_hbm, dtab_hbm, rtab_hbm,
                      dout_hbm, rout_hbm,
                      didx_v, ridx_v, drows_v, rrows_v, sem):
        wid = lax.axis_index("s") * NC + lax.axis_index("c")
        rowbase = wid * n_chunks
        pltpu.sync_copy(didx_hbm.at[pl.ds(rowbase, n_chunks)], didx_v)
        pltpu.sync_copy(ridx_hbm.at[pl.ds(rowbase, n_chunks)], ridx_v)
        copies = []
        for j in range(n_chunks):
            copies.append(pltpu.async_copy(
                dtab_hbm.at[didx_v.at[j]],
                drows_v.at[pl.ds(j * _CHUNK, _CHUNK)], sem))
        for j in range(n_chunks):
            copies.append(pltpu.async_copy(
                rtab_hbm.at[ridx_v.at[j]],
                rrows_v.at[pl.ds(j * _CHUNK, _CHUNK)], sem))
        for cp in copies:
            cp.wait()
        base = wid * b_per_w
        pltpu.sync_copy(drows_v, dout_hbm.at[pl.ds(base, b_per_w)])
        pltpu.sync_copy(rrows_v, rout_hbm.at[pl.ds(base, b_per_w)])

    return gather_kernel(didx2, ridx2, dtab16, rtab16)


def _mlp_body(dr, rr, age, sx, rt, stab, rtab,
              w1a, w1b, w1age, w1d, w1e, b1, w2, b2, w3, b3, out):
    TB = dr.shape[0]
    f32 = jnp.float32
    acc = jnp.dot(dr[...], w1a[...], preferred_element_type=f32)
    acc = acc + jnp.dot(rr[...], w1b[...], preferred_element_type=f32)
    acc = acc + age[...] * w1age[...]
    n_sex = stab.shape[0]
    soh = (sx[...] == lax.broadcasted_iota(jnp.int32, (TB, n_sex), 1)
           ).astype(f32)
    sproj = jnp.dot(stab[...], w1d[...], preferred_element_type=f32)
    acc = acc + jnp.dot(soh, sproj, preferred_element_type=f32)
    n_route = rtab.shape[0]
    roh = (rt[...] == lax.broadcasted_iota(jnp.int32, (TB, n_route), 1)
           ).astype(f32)
    rproj = jnp.dot(rtab[...], w1e[...], preferred_element_type=f32)
    acc = acc + jnp.dot(roh, rproj, preferred_element_type=f32)
    h = jnp.maximum(acc + b1[...], 0.0)
    h = jnp.maximum(jnp.dot(h, w2[...], preferred_element_type=f32)
                    + b2[...], 0.0)
    o = jnp.dot(h, w3[...], preferred_element_type=f32) + b3[...]
    out[...] = jax.nn.sigmoid(o)


def kernel(drug_indices, reaction_indices, age, sex_indices, route_indices,
           drug_table, reaction_table, sex_table, route_table,
           W1, b1, W2, b2, W3, b3):
    B = drug_indices.shape[0]
    EMB = drug_table.shape[1]
    SEX_EMB = sex_table.shape[1]
    ROUTE_EMB = route_table.shape[1]
    H1 = W1.shape[1]
    H2 = W2.shape[1]

    bf16 = jnp.bfloat16
    drug_rows, reaction_rows = _sc_gather(
        drug_indices, reaction_indices,
        drug_table.astype(bf16), reaction_table.astype(bf16))

    w1a = W1[:EMB].astype(bf16)
    w1b = W1[EMB:2 * EMB].astype(bf16)
    w1age = W1[2 * EMB:2 * EMB + 1]
    w1d = W1[2 * EMB + 1:2 * EMB + 1 + SEX_EMB]
    w1e = W1[2 * EMB + 1 + SEX_EMB:]

    TB = 2048
    grid = (B // TB,)

    def blk(shape):
        return pl.BlockSpec(shape, lambda i: (0,) * len(shape))

    out = pl.pallas_call(
        _mlp_body,
        grid=grid,
        in_specs=[
            pl.BlockSpec((TB, EMB), lambda i: (i, 0)),
            pl.BlockSpec((TB, EMB), lambda i: (i, 0)),
            pl.BlockSpec((TB, 1), lambda i: (i, 0)),
            pl.BlockSpec((TB, 1), lambda i: (i, 0)),
            pl.BlockSpec((TB, 1), lambda i: (i, 0)),
            blk(sex_table.shape),
            blk(route_table.shape),
            blk((EMB, H1)),
            blk((EMB, H1)),
            blk((1, H1)),
            blk((SEX_EMB, H1)),
            blk((ROUTE_EMB, H1)),
            blk((1, H1)),
            blk((H1, H2)),
            blk((1, H2)),
            blk((H2, 1)),
            blk((1, 1)),
        ],
        out_specs=pl.BlockSpec((TB, 1), lambda i: (i, 0)),
        out_shape=jax.ShapeDtypeStruct((B, 1), jnp.float32),
    )(drug_rows, reaction_rows,
      age.reshape(B, 1), sex_indices.reshape(B, 1),
      route_indices.reshape(B, 1),
      sex_table, route_table,
      w1a, w1b, w1age, w1d, w1e,
      b1.reshape(1, H1), W2, b2.reshape(1, H2), W3, b3.reshape(1, 1))
    return out


# speedup vs baseline: 2.0828x; 2.0828x over previous
import functools

import jax
import jax.numpy as jnp
from jax import lax
from jax.experimental import pallas as pl
from jax.experimental.pallas import tpu as pltpu
from jax.experimental.pallas import tpu_sc as plsc

_CHUNK = 128
_PACK_V = 2048


def _pack_body(top, bot, out):
    t = jnp.transpose(top[...])
    b = jnp.transpose(bot[...])
    out[...] = jnp.concatenate([t, b], axis=1)


def _pack_table(tab_t):
    EMB, N = tab_t.shape
    V = _PACK_V
    nb_arr = -(-N // V)
    nb_out = -(-nb_arr // 2)
    S = (nb_arr - nb_out) * V if nb_arr > 1 else V
    H = nb_out * V
    bot0 = S // V

    packed = pl.pallas_call(
        _pack_body,
        grid=(nb_out,),
        in_specs=[
            pl.BlockSpec((EMB, V), lambda i: (0, i)),
            pl.BlockSpec(
                (EMB, V),
                lambda i: (0, jnp.minimum(bot0 + i, nb_arr - 1))),
        ],
        out_specs=pl.BlockSpec((V, 2 * EMB), lambda i: (i, 0)),
        out_shape=jax.ShapeDtypeStruct((H, 2 * EMB), jnp.float32),
    )(tab_t, tab_t)
    return packed, S


def _sc_gather(didx, ridx, dpack, rpack, sd, sr):
    B = didx.shape[0]
    PW = dpack.shape[1]
    info = plsc.get_sparse_core_info()
    NC, NS, L = info.num_cores, info.num_subcores, info.num_lanes
    NW = NC * NS
    b_per_w = B // NW
    n_chunks = b_per_w // _CHUNK

    didx2 = didx.reshape(B // _CHUNK, _CHUNK)
    ridx2 = ridx.reshape(B // _CHUNK, _CHUNK)

    mesh = plsc.VectorSubcoreMesh(core_axis_name="c", subcore_axis_name="s")

    @functools.partial(
        pl.kernel,
        mesh=mesh,
        out_type=[
            jax.ShapeDtypeStruct((B, PW), jnp.float32),
            jax.ShapeDtypeStruct((B, PW), jnp.float32),
        ],
        scratch_types=[
            pltpu.VMEM((n_chunks, _CHUNK), jnp.int32),
            pltpu.VMEM((n_chunks, _CHUNK), jnp.int32),
            pltpu.VMEM((b_per_w // 2, PW), jnp.float32),
            pltpu.VMEM((b_per_w // 2, PW), jnp.float32),
            pltpu.SemaphoreType.DMA,
        ],
    )
    def gather_kernel(didx_hbm, ridx_hbm, dpack_hbm, rpack_hbm,
                      dout_hbm, rout_hbm,
                      didx_v, ridx_v, drows_v, rrows_v, sem):
        wid = lax.axis_index("s") * NC + lax.axis_index("c")
        rowbase = wid * n_chunks
        pltpu.sync_copy(didx_hbm.at[pl.ds(rowbase, n_chunks)], didx_v)
        pltpu.sync_copy(ridx_hbm.at[pl.ds(rowbase, n_chunks)], ridx_v)

        for j in range(n_chunks):
            for k in range(_CHUNK // L):
                dvec = didx_v[j, pl.ds(k * L, L)]
                didx_v[j, pl.ds(k * L, L)] = jnp.where(
                    dvec >= sd, dvec - sd, dvec)
                rvec = ridx_v[j, pl.ds(k * L, L)]
                ridx_v[j, pl.ds(k * L, L)] = jnp.where(
                    rvec >= sr, rvec - sr, rvec)

        half_chunks = n_chunks // 2
        for p in range(2):
            copies = []
            for j in range(half_chunks):
                copies.append(pltpu.async_copy(
                    dpack_hbm.at[didx_v.at[p * half_chunks + j]],
                    drows_v.at[pl.ds(j * _CHUNK, _CHUNK)], sem))
                copies.append(pltpu.async_copy(
                    rpack_hbm.at[ridx_v.at[p * half_chunks + j]],
                    rrows_v.at[pl.ds(j * _CHUNK, _CHUNK)], sem))
            for cp in copies:
                cp.wait()
            base = wid * b_per_w + p * (b_per_w // 2)
            pltpu.sync_copy(drows_v, dout_hbm.at[pl.ds(base, b_per_w // 2)])
            pltpu.sync_copy(rrows_v, rout_hbm.at[pl.ds(base, b_per_w // 2)])

    return gather_kernel(didx2, ridx2, dpack, rpack)


def _make_mlp_body(sd, sr, emb):
    def _mlp_body(dr2, rr2, didx, ridx, age, sx, rt, stab, rtab,
                  w1a, w1b, w1age, w1d, w1e, b1, w2, b2, w3, b3, out):
        TB = dr2.shape[0]
        f32 = jnp.float32
        dr = jnp.where(didx[...] >= sd, dr2[:, emb:], dr2[:, :emb])
        rr = jnp.where(ridx[...] >= sr, rr2[:, emb:], rr2[:, :emb])
        acc = jnp.dot(dr, w1a[...], preferred_element_type=f32)
        acc = acc + jnp.dot(rr, w1b[...], preferred_element_type=f32)
        acc = acc + age[...] * w1age[...]
        n_sex = stab.shape[0]
        soh = (sx[...] == lax.broadcasted_iota(jnp.int32, (TB, n_sex), 1)
               ).astype(f32)
        sproj = jnp.dot(stab[...], w1d[...], preferred_element_type=f32)
        acc = acc + jnp.dot(soh, sproj, preferred_element_type=f32)
        n_route = rtab.shape[0]
        roh = (rt[...] == lax.broadcasted_iota(jnp.int32, (TB, n_route), 1)
               ).astype(f32)
        rproj = jnp.dot(rtab[...], w1e[...], preferred_element_type=f32)
        acc = acc + jnp.dot(roh, rproj, preferred_element_type=f32)
        h = jnp.maximum(acc + b1[...], 0.0)
        h = jnp.maximum(jnp.dot(h, w2[...], preferred_element_type=f32)
                        + b2[...], 0.0)
        o = jnp.dot(h, w3[...], preferred_element_type=f32) + b3[...]
        out[...] = jax.nn.sigmoid(o)
    return _mlp_body


def kernel(drug_indices, reaction_indices, age, sex_indices, route_indices,
           drug_table, reaction_table, sex_table, route_table,
           W1, b1, W2, b2, W3, b3):
    B = drug_indices.shape[0]
    EMB = drug_table.shape[1]
    SEX_EMB = sex_table.shape[1]
    ROUTE_EMB = route_table.shape[1]
    H1 = W1.shape[1]
    H2 = W2.shape[1]

    dpack, sd = _pack_table(drug_table.T)
    rpack, sr = _pack_table(reaction_table.T)
    drug_rows2, reaction_rows2 = _sc_gather(
        drug_indices, reaction_indices, dpack, rpack, sd, sr)

    w1a = W1[:EMB]
    w1b = W1[EMB:2 * EMB]
    w1age = W1[2 * EMB:2 * EMB + 1]
    w1d = W1[2 * EMB + 1:2 * EMB + 1 + SEX_EMB]
    w1e = W1[2 * EMB + 1 + SEX_EMB:]

    TB = 2048
    grid = (B // TB,)

    def blk(shape):
        return pl.BlockSpec(shape, lambda i: (0,) * len(shape))

    out = pl.pallas_call(
        _make_mlp_body(sd, sr, EMB),
        grid=grid,
        in_specs=[
            pl.BlockSpec((TB, 2 * EMB), lambda i: (i, 0)),
            pl.BlockSpec((TB, 2 * EMB), lambda i: (i, 0)),
            pl.BlockSpec((TB, 1), lambda i: (i, 0)),
            pl.BlockSpec((TB, 1), lambda i: (i, 0)),
            pl.BlockSpec((TB, 1), lambda i: (i, 0)),
            pl.BlockSpec((TB, 1), lambda i: (i, 0)),
            pl.BlockSpec((TB, 1), lambda i: (i, 0)),
            blk(sex_table.shape),
            blk(route_table.shape),
            blk((EMB, H1)),
            blk((EMB, H1)),
            blk((1, H1)),
            blk((SEX_EMB, H1)),
            blk((ROUTE_EMB, H1)),
            blk((1, H1)),
            blk((H1, H2)),
            blk((1, H2)),
            blk((H2, 1)),
            blk((1, 1)),
        ],
        out_specs=pl.BlockSpec((TB, 1), lambda i: (i, 0)),
        out_shape=jax.ShapeDtypeStruct((B, 1), jnp.float32),
    )(drug_rows2, reaction_rows2,
      drug_indices.reshape(B, 1), reaction_indices.reshape(B, 1),
      age.reshape(B, 1), sex_indices.reshape(B, 1),
      route_indices.reshape(B, 1),
      sex_table, route_table,
      w1a, w1b, w1age, w1d, w1e,
      b1.reshape(1, H1), W2, b2.reshape(1, H2), W3, b3.reshape(1, 1))
    return out


# speedup vs baseline: 2.0922x; 1.0045x over previous
import functools

import jax
import jax.numpy as jnp
from jax import lax
from jax.experimental import pallas as pl
from jax.experimental.pallas import tpu as pltpu
from jax.experimental.pallas import tpu_sc as plsc

_CHUNK = 128
_PACK_V = 2048


def _pack_body(top, bot, out):
    def halfpack(x):
        h = x.shape[0] // 2

        def bf16_bits(v):
            r = v.astype(jnp.bfloat16).astype(jnp.float32)
            return lax.bitcast_convert_type(r, jnp.uint32)

        lo = lax.shift_right_logical(bf16_bits(x[:h, :]), jnp.uint32(16))
        hi = bf16_bits(x[h:, :])
        return jnp.transpose(
            lax.bitcast_convert_type(lo | hi, jnp.int32))

    out[...] = jnp.concatenate(
        [halfpack(top[...]), halfpack(bot[...])], axis=1)


def _pack_table(tab_t):
    EMB, N = tab_t.shape
    V = _PACK_V
    nb_arr = -(-N // V)
    nb_out = -(-nb_arr // 2)
    S = (nb_arr - nb_out) * V if nb_arr > 1 else V
    H = nb_out * V
    bot0 = S // V

    packed = pl.pallas_call(
        _pack_body,
        grid=(nb_out,),
        in_specs=[
            pl.BlockSpec((EMB, V), lambda i: (0, i)),
            pl.BlockSpec(
                (EMB, V),
                lambda i: (0, jnp.minimum(bot0 + i, nb_arr - 1))),
        ],
        out_specs=pl.BlockSpec((V, EMB), lambda i: (i, 0)),
        out_shape=jax.ShapeDtypeStruct((H, EMB), jnp.int32),
    )(tab_t, tab_t)
    return packed, S


def _sc_gather(didx, ridx, dpack, rpack, sd, sr):
    B = didx.shape[0]
    PW = dpack.shape[1]
    info = plsc.get_sparse_core_info()
    NC, NS, L = info.num_cores, info.num_subcores, info.num_lanes
    NW = NC * NS
    b_per_w = B // NW
    n_chunks = b_per_w // _CHUNK

    didx2 = didx.reshape(B // _CHUNK, _CHUNK)
    ridx2 = ridx.reshape(B // _CHUNK, _CHUNK)

    mesh = plsc.VectorSubcoreMesh(core_axis_name="c", subcore_axis_name="s")

    @functools.partial(
        pl.kernel,
        mesh=mesh,
        out_type=[
            jax.ShapeDtypeStruct((B, PW), jnp.int32),
            jax.ShapeDtypeStruct((B, PW), jnp.int32),
        ],
        scratch_types=[
            pltpu.VMEM((n_chunks, _CHUNK), jnp.int32),
            pltpu.VMEM((n_chunks, _CHUNK), jnp.int32),
            pltpu.VMEM((b_per_w // 2, PW), jnp.int32),
            pltpu.VMEM((b_per_w // 2, PW), jnp.int32),
            pltpu.SemaphoreType.DMA,
        ],
    )
    def gather_kernel(didx_hbm, ridx_hbm, dpack_hbm, rpack_hbm,
                      dout_hbm, rout_hbm,
                      didx_v, ridx_v, drows_v, rrows_v, sem):
        wid = lax.axis_index("s") * NC + lax.axis_index("c")
        rowbase = wid * n_chunks
        pltpu.sync_copy(didx_hbm.at[pl.ds(rowbase, n_chunks)], didx_v)
        pltpu.sync_copy(ridx_hbm.at[pl.ds(rowbase, n_chunks)], ridx_v)

        for j in range(n_chunks):
            for k in range(_CHUNK // L):
                dvec = didx_v[j, pl.ds(k * L, L)]
                didx_v[j, pl.ds(k * L, L)] = jnp.where(
                    dvec >= sd, dvec - sd, dvec)
                rvec = ridx_v[j, pl.ds(k * L, L)]
                ridx_v[j, pl.ds(k * L, L)] = jnp.where(
                    rvec >= sr, rvec - sr, rvec)

        half_steps = (b_per_w // 2) // L
        for p in range(2):
            def step(c, _, p=p):
                cc = p * half_steps + c
                j = cc // (_CHUNK // L)
                col = (cc % (_CHUNK // L)) * L
                dvec = didx_v[j, pl.ds(col, L)]
                rvec = ridx_v[j, pl.ds(col, L)]
                base = c * L
                for lane in range(L):
                    pltpu.async_copy(
                        dpack_hbm.at[pl.ds(dvec[lane], 1)],
                        drows_v.at[pl.ds(base + lane, 1)], sem)
                    pltpu.async_copy(
                        rpack_hbm.at[pl.ds(rvec[lane], 1)],
                        rrows_v.at[pl.ds(base + lane, 1)], sem)
                @pl.when(c > 0)
                def _():
                    pltpu.make_async_copy(
                        dpack_hbm.at[pl.ds(0, L)],
                        drows_v.at[pl.ds(base - L, L)], sem).wait()
                    pltpu.make_async_copy(
                        rpack_hbm.at[pl.ds(0, L)],
                        rrows_v.at[pl.ds(base - L, L)], sem).wait()
                return 0

            lax.fori_loop(0, half_steps, step, 0)
            last = (half_steps - 1) * L
            pltpu.make_async_copy(dpack_hbm.at[pl.ds(0, L)],
                                  drows_v.at[pl.ds(last, L)], sem).wait()
            pltpu.make_async_copy(rpack_hbm.at[pl.ds(0, L)],
                                  rrows_v.at[pl.ds(last, L)], sem).wait()
            base = wid * b_per_w + p * (b_per_w // 2)
            pltpu.sync_copy(drows_v, dout_hbm.at[pl.ds(base, b_per_w // 2)])
            pltpu.sync_copy(rrows_v, rout_hbm.at[pl.ds(base, b_per_w // 2)])

    return gather_kernel(didx2, ridx2, dpack, rpack)


def _make_mlp_body(sd, sr, emb):
    def _mlp_body(dr2, rr2, didx, ridx, age, sx, rt, stab, rtab,
                  w1a, w1b, w1age, w1d, w1e, b1, w2, b2, w3, b3, out):
        TB = dr2.shape[0]
        f32 = jnp.float32
        h2e = emb // 2
        dsel = jnp.where(didx[...] >= sd, dr2[:, h2e:], dr2[:, :h2e])
        rsel = jnp.where(ridx[...] >= sr, rr2[:, h2e:], rr2[:, :h2e])

        def unpack_lo(x):
            return lax.bitcast_convert_type(
                lax.shift_left(x, 16), jnp.float32)

        def unpack_hi(x):
            return lax.bitcast_convert_type(
                lax.bitwise_and(x, jnp.int32(-65536)), jnp.float32)

        w1af = w1a[...]
        w1bf = w1b[...]
        acc = jnp.dot(unpack_lo(dsel), w1af[:h2e],
                      preferred_element_type=f32)
        acc = acc + jnp.dot(unpack_hi(dsel), w1af[h2e:],
                            preferred_element_type=f32)
        acc = acc + jnp.dot(unpack_lo(rsel), w1bf[:h2e],
                            preferred_element_type=f32)
        acc = acc + jnp.dot(unpack_hi(rsel), w1bf[h2e:],
                            preferred_element_type=f32)
        acc = acc + age[...] * w1age[...]
        n_sex = stab.shape[0]
        soh = (sx[...] == lax.broadcasted_iota(jnp.int32, (TB, n_sex), 1)
               ).astype(f32)
        sproj = jnp.dot(stab[...], w1d[...], preferred_element_type=f32)
        acc = acc + jnp.dot(soh, sproj, preferred_element_type=f32)
        n_route = rtab.shape[0]
        roh = (rt[...] == lax.broadcasted_iota(jnp.int32, (TB, n_route), 1)
               ).astype(f32)
        rproj = jnp.dot(rtab[...], w1e[...], preferred_element_type=f32)
        acc = acc + jnp.dot(roh, rproj, preferred_element_type=f32)
        h = jnp.maximum(acc + b1[...], 0.0)
        h = jnp.maximum(jnp.dot(h, w2[...], preferred_element_type=f32)
                        + b2[...], 0.0)
        o = jnp.dot(h, w3[...], preferred_element_type=f32) + b3[...]
        out[...] = jax.nn.sigmoid(o)
    return _mlp_body


def kernel(drug_indices, reaction_indices, age, sex_indices, route_indices,
           drug_table, reaction_table, sex_table, route_table,
           W1, b1, W2, b2, W3, b3):
    B = drug_indices.shape[0]
    EMB = drug_table.shape[1]
    SEX_EMB = sex_table.shape[1]
    ROUTE_EMB = route_table.shape[1]
    H1 = W1.shape[1]
    H2 = W2.shape[1]

    dpack, sd = _pack_table(drug_table.T)
    rpack, sr = _pack_table(reaction_table.T)
    drug_rows2, reaction_rows2 = _sc_gather(
        drug_indices, reaction_indices, dpack, rpack, sd, sr)

    w1a = W1[:EMB]
    w1b = W1[EMB:2 * EMB]
    w1age = W1[2 * EMB:2 * EMB + 1]
    w1d = W1[2 * EMB + 1:2 * EMB + 1 + SEX_EMB]
    w1e = W1[2 * EMB + 1 + SEX_EMB:]

    TB = 2048
    grid = (B // TB,)

    def blk(shape):
        return pl.BlockSpec(shape, lambda i: (0,) * len(shape))

    out = pl.pallas_call(
        _make_mlp_body(sd, sr, EMB),
        grid=grid,
        in_specs=[
            pl.BlockSpec((TB, EMB), lambda i: (i, 0)),
            pl.BlockSpec((TB, EMB), lambda i: (i, 0)),
            pl.BlockSpec((TB, 1), lambda i: (i, 0)),
            pl.BlockSpec((TB, 1), lambda i: (i, 0)),
            pl.BlockSpec((TB, 1), lambda i: (i, 0)),
            pl.BlockSpec((TB, 1), lambda i: (i, 0)),
            pl.BlockSpec((TB, 1), lambda i: (i, 0)),
            blk(sex_table.shape),
            blk(route_table.shape),
            blk((EMB, H1)),
            blk((EMB, H1)),
            blk((1, H1)),
            blk((SEX_EMB, H1)),
            blk((ROUTE_EMB, H1)),
            blk((1, H1)),
            blk((H1, H2)),
            blk((1, H2)),
            blk((H2, 1)),
            blk((1, 1)),
        ],
        out_specs=pl.BlockSpec((TB, 1), lambda i: (i, 0)),
        out_shape=jax.ShapeDtypeStruct((B, 1), jnp.float32),
    )(drug_rows2, reaction_rows2,
      drug_indices.reshape(B, 1), reaction_indices.reshape(B, 1),
      age.reshape(B, 1), sex_indices.reshape(B, 1),
      route_indices.reshape(B, 1),
      sex_table, route_table,
      w1a, w1b, w1age, w1d, w1e,
      b1.reshape(1, H1), W2, b2.reshape(1, H2), W3, b3.reshape(1, 1))
    return out


# speedup vs baseline: 2.1019x; 1.0046x over previous
import functools

import jax
import jax.numpy as jnp
from jax import lax
from jax.experimental import pallas as pl
from jax.experimental.pallas import tpu as pltpu
from jax.experimental.pallas import tpu_sc as plsc

_CHUNK = 128


def _sc_gather(didx, ridx, drug_table, reaction_table):
    B = didx.shape[0]
    EMB = drug_table.shape[1]
    info = plsc.get_sparse_core_info()
    NC, NS, L = info.num_cores, info.num_subcores, info.num_lanes
    NW = NC * NS
    b_per_w = B // NW
    n_steps = b_per_w // L

    didx2 = didx.reshape(B // _CHUNK, _CHUNK)
    ridx2 = ridx.reshape(B // _CHUNK, _CHUNK)
    rows_per_w = b_per_w // _CHUNK

    mesh = plsc.VectorSubcoreMesh(core_axis_name="c", subcore_axis_name="s")

    @functools.partial(
        pl.kernel,
        mesh=mesh,
        out_type=[
            jax.ShapeDtypeStruct((B, EMB), jnp.float32),
            jax.ShapeDtypeStruct((B, EMB), jnp.float32),
        ],
        scratch_types=[
            pltpu.VMEM((rows_per_w, _CHUNK), jnp.int32),
            pltpu.VMEM((rows_per_w, _CHUNK), jnp.int32),
            pltpu.VMEM((b_per_w // 2, EMB), jnp.float32),
            pltpu.VMEM((b_per_w // 2, EMB), jnp.float32),
            pltpu.SemaphoreType.DMA,
        ],
    )
    def gather_kernel(didx_hbm, ridx_hbm, dtab_hbm, rtab_hbm,
                      dout_hbm, rout_hbm,
                      didx_v, ridx_v, drows_v, rrows_v, sem):
        wid = lax.axis_index("s") * NC + lax.axis_index("c")
        rowbase = wid * rows_per_w
        pltpu.sync_copy(didx_hbm.at[pl.ds(rowbase, rows_per_w)], didx_v)
        pltpu.sync_copy(ridx_hbm.at[pl.ds(rowbase, rows_per_w)], ridx_v)

        half_steps = n_steps // 2
        for p in range(2):
            def step(c, _, p=p):
                cc = p * half_steps + c
                j = cc // (_CHUNK // L)
                col = (cc % (_CHUNK // L)) * L
                dvec = didx_v[j, pl.ds(col, L)]
                rvec = ridx_v[j, pl.ds(col, L)]
                base = c * L
                for lane in range(L):
                    pltpu.async_copy(
                        dtab_hbm.at[pl.ds(dvec[lane], 1)],
                        drows_v.at[pl.ds(base + lane, 1)], sem)
                    pltpu.async_copy(
                        rtab_hbm.at[pl.ds(rvec[lane], 1)],
                        rrows_v.at[pl.ds(base + lane, 1)], sem)
                @pl.when(c > 0)
                def _():
                    pltpu.make_async_copy(
                        dtab_hbm.at[pl.ds(0, L)],
                        drows_v.at[pl.ds(base - L, L)], sem).wait()
                    pltpu.make_async_copy(
                        rtab_hbm.at[pl.ds(0, L)],
                        rrows_v.at[pl.ds(base - L, L)], sem).wait()
                return 0

            lax.fori_loop(0, half_steps, step, 0)
            last = (half_steps - 1) * L
            pltpu.make_async_copy(dtab_hbm.at[pl.ds(0, L)],
                                  drows_v.at[pl.ds(last, L)], sem).wait()
            pltpu.make_async_copy(rtab_hbm.at[pl.ds(0, L)],
                                  rrows_v.at[pl.ds(last, L)], sem).wait()

            base = wid * b_per_w + p * (b_per_w // 2)
            pltpu.sync_copy(drows_v, dout_hbm.at[pl.ds(base, b_per_w // 2)])
            pltpu.sync_copy(rrows_v, rout_hbm.at[pl.ds(base, b_per_w // 2)])

    return gather_kernel(didx2, ridx2, drug_table, reaction_table)


def _mlp_body(dr, rr, age, sx, rt, stab, rtab,
              w1a, w1b, w1age, w1d, w1e, b1, w2, b2, w3, b3, out):
    TB = dr.shape[0]
    f32 = jnp.float32
    acc = jnp.dot(dr[...], w1a[...], preferred_element_type=f32)
    acc = acc + jnp.dot(rr[...], w1b[...], preferred_element_type=f32)
    acc = acc + age[...] * w1age[...]
    n_sex = stab.shape[0]
    soh = (sx[...] == lax.broadcasted_iota(jnp.int32, (TB, n_sex), 1)
           ).astype(f32)
    sproj = jnp.dot(stab[...], w1d[...], preferred_element_type=f32)
    acc = acc + jnp.dot(soh, sproj, preferred_element_type=f32)
    n_route = rtab.shape[0]
    roh = (rt[...] == lax.broadcasted_iota(jnp.int32, (TB, n_route), 1)
           ).astype(f32)
    rproj = jnp.dot(rtab[...], w1e[...], preferred_element_type=f32)
    acc = acc + jnp.dot(roh, rproj, preferred_element_type=f32)
    h = jnp.maximum(acc + b1[...], 0.0)
    h = jnp.maximum(jnp.dot(h, w2[...], preferred_element_type=f32)
                    + b2[...], 0.0)
    o = jnp.dot(h, w3[...], preferred_element_type=f32) + b3[...]
    out[...] = jax.nn.sigmoid(o)


def kernel(drug_indices, reaction_indices, age, sex_indices, route_indices,
           drug_table, reaction_table, sex_table, route_table,
           W1, b1, W2, b2, W3, b3):
    B = drug_indices.shape[0]
    EMB = drug_table.shape[1]
    SEX_EMB = sex_table.shape[1]
    ROUTE_EMB = route_table.shape[1]
    H1 = W1.shape[1]
    H2 = W2.shape[1]

    drug_rows, reaction_rows = _sc_gather(
        drug_indices, reaction_indices, drug_table, reaction_table)

    w1a = W1[:EMB]
    w1b = W1[EMB:2 * EMB]
    w1age = W1[2 * EMB:2 * EMB + 1]
    w1d = W1[2 * EMB + 1:2 * EMB + 1 + SEX_EMB]
    w1e = W1[2 * EMB + 1 + SEX_EMB:]

    TB = 2048
    grid = (B // TB,)

    def blk(shape):
        return pl.BlockSpec(shape, lambda i: (0,) * len(shape))

    out = pl.pallas_call(
        _mlp_body,
        grid=grid,
        in_specs=[
            pl.BlockSpec((TB, EMB), lambda i: (i, 0)),
            pl.BlockSpec((TB, EMB), lambda i: (i, 0)),
            pl.BlockSpec((TB, 1), lambda i: (i, 0)),
            pl.BlockSpec((TB, 1), lambda i: (i, 0)),
            pl.BlockSpec((TB, 1), lambda i: (i, 0)),
            blk(sex_table.shape),
            blk(route_table.shape),
            blk((EMB, H1)),
            blk((EMB, H1)),
            blk((1, H1)),
            blk((SEX_EMB, H1)),
            blk((ROUTE_EMB, H1)),
            blk((1, H1)),
            blk((H1, H2)),
            blk((1, H2)),
            blk((H2, 1)),
            blk((1, 1)),
        ],
        out_specs=pl.BlockSpec((TB, 1), lambda i: (i, 0)),
        out_shape=jax.ShapeDtypeStruct((B, 1), jnp.float32),
    )(drug_rows, reaction_rows,
      age.reshape(B, 1), sex_indices.reshape(B, 1),
      route_indices.reshape(B, 1),
      sex_table, route_table,
      w1a, w1b, w1age, w1d, w1e,
      b1.reshape(1, H1), W2, b2.reshape(1, H2), W3, b3.reshape(1, 1))
    return out
